# Initial kernel scaffold; baseline (speedup 1.0000x reference)
import jax
import jax.numpy as jnp
from jax.experimental import pallas as pl


def kernel(x, in_vertices, out_vertices, W_pool, b_pool, W_conv, b_conv, gamma, beta):
    raise NotImplementedError("write your pallas kernel here")


# speedup vs baseline: 2.1588x; 2.1588x over previous
import functools

import jax
import jax.numpy as jnp
from jax import lax
from jax.experimental import pallas as pl
from jax.experimental.pallas import tpu as pltpu
from jax.experimental.pallas import tpu_sc as plsc

B = 4
C = 128
N_IN = 50000
N_OUT = 12500
K = 16
SIGMA = 0.1
EPS_BN = 1e-5

N_IN_PAD = 50048
N_OUT_PAD = 12544
QBLK = 128
NQB = N_OUT_PAD // QBLK
ROWS_PER_W = N_OUT_PAD // 32
RBLK = 8
NRB = ROWS_PER_W // RBLK


def _knn_body(qx, qy, qz, px, py, pz, idx_ref, w_ref, dref):
  d = (qx[...] - px[...]) ** 2 + (qy[...] - py[...]) ** 2 + (qz[...] - pz[...]) ** 2
  dref[...] = d
  lane = lax.broadcasted_iota(jnp.int32, (QBLK, N_IN_PAD), 1)
  vals, inds = [], []
  for _ in range(K):
    d = dref[...]
    m = jnp.min(d, axis=1, keepdims=True)
    a = jnp.argmin(d, axis=1).astype(jnp.int32)
    vals.append(m)
    inds.append(a[:, None])
    dref[...] = jnp.where(lane == a[:, None], jnp.float32(jnp.inf), d)
  idx_ref[...] = jnp.concatenate(inds, axis=1)
  d2 = jnp.concatenate(vals, axis=1)
  wk = jnp.exp(-d2 / (2.0 * SIGMA * SIGMA))
  w_ref[...] = wk / (jnp.sum(wk, axis=1, keepdims=True) + 1e-12)


def _knn(qx, qy, qz, px, py, pz):
  return pl.pallas_call(
      _knn_body,
      grid=(NQB,),
      in_specs=[
          pl.BlockSpec((QBLK, 1), lambda i: (i, 0)),
          pl.BlockSpec((QBLK, 1), lambda i: (i, 0)),
          pl.BlockSpec((QBLK, 1), lambda i: (i, 0)),
          pl.BlockSpec((1, N_IN_PAD), lambda i: (0, 0)),
          pl.BlockSpec((1, N_IN_PAD), lambda i: (0, 0)),
          pl.BlockSpec((1, N_IN_PAD), lambda i: (0, 0)),
      ],
      out_specs=[
          pl.BlockSpec((QBLK, K), lambda i: (i, 0)),
          pl.BlockSpec((QBLK, K), lambda i: (i, 0)),
      ],
      out_shape=[
          jax.ShapeDtypeStruct((N_OUT_PAD, K), jnp.int32),
          jax.ShapeDtypeStruct((N_OUT_PAD, K), jnp.float32),
      ],
      scratch_shapes=[pltpu.VMEM((QBLK, N_IN_PAD), jnp.float32)],
  )(qx, qy, qz, px, py, pz)


def _sc_gather_pool(xT, idx, wexp):
  mesh = plsc.VectorSubcoreMesh(core_axis_name="c", subcore_axis_name="s")

  @functools.partial(
      pl.kernel,
      mesh=mesh,
      out_type=[
          jax.ShapeDtypeStruct((B, N_OUT_PAD, C), jnp.float32),
          jax.ShapeDtypeStruct((B, N_OUT_PAD, C), jnp.float32),
      ],
      scratch_types=[
          pltpu.VMEM((ROWS_PER_W, K), jnp.int32),
          pltpu.VMEM((RBLK, K * 16), jnp.float32),
          pltpu.VMEM((B, K, C), jnp.float32),
          pltpu.VMEM((B, RBLK, C), jnp.float32),
          pltpu.VMEM((B, RBLK, C), jnp.float32),
          pltpu.SemaphoreType.DMA,
      ],
  )
  def kern(xT_hbm, idx_hbm, w_hbm, pooled_hbm, agg_hbm,
           idx_v, w_v, rows_v, poolblk, aggblk, sem):
    wid = lax.axis_index("s") * 2 + lax.axis_index("c")
    base = wid * ROWS_PER_W
    pltpu.sync_copy(idx_hbm.at[pl.ds(base, ROWS_PER_W)], idx_v)

    def do_block(blk, _):
      pltpu.sync_copy(w_hbm.at[pl.ds(base + blk * RBLK, RBLK)], w_v)

      def do_row(rr, _):
        r = blk * RBLK + rr
        iv = idx_v[r]
        cps = []
        for b in range(B):
          cps.append(pltpu.async_copy(
              xT_hbm.at[iv + b * N_IN_PAD], rows_v.at[b], sem))
        for cp in cps:
          cp.wait()
        wks = [w_v[rr, pl.ds(kk * 16, 16)] for kk in range(K)]
        for b in range(B):
          for g in range(C // 16):
            accp = rows_v[b, 0, pl.ds(g * 16, 16)]
            acca = wks[0] * accp
            for kk in range(1, K):
              row = rows_v[b, kk, pl.ds(g * 16, 16)]
              accp = accp + row
              acca = acca + wks[kk] * row
            poolblk[b, rr, pl.ds(g * 16, 16)] = accp * jnp.float32(1.0 / K)
            aggblk[b, rr, pl.ds(g * 16, 16)] = acca
        return _

      lax.fori_loop(0, RBLK, do_row, 0)
      for b in range(B):
        pltpu.sync_copy(poolblk.at[b],
                        pooled_hbm.at[b, pl.ds(base + blk * RBLK, RBLK)])
        pltpu.sync_copy(aggblk.at[b],
                        agg_hbm.at[b, pl.ds(base + blk * RBLK, RBLK)])
      return _

    lax.fori_loop(0, NRB, do_block, 0)

  return kern(xT, idx, wexp)


def _conv_body(pooled_ref, agg_ref, wp_ref, wc_ref, bp_ref, bc_ref,
               ident_ref, y_ref, ssum_ref, ssq_ref):
  i = pl.program_id(0)

  @pl.when(i == 0)
  def _():
    ssum_ref[...] = jnp.zeros((C, 1), jnp.float32)
    ssq_ref[...] = jnp.zeros((C, 1), jnp.float32)

  valid = (lax.broadcasted_iota(jnp.int32, (1, QBLK), 1) + i * QBLK) < N_OUT
  s = jnp.zeros((C, 1), jnp.float32)
  q = jnp.zeros((C, 1), jnp.float32)
  dn = (((1,), (1,)), ((), ()))
  for b in range(B):
    ib = lax.dot_general(wp_ref[...], pooled_ref[b], dn,
                         preferred_element_type=jnp.float32) + bp_ref[...]
    ident_ref[b] = ib
    yb = lax.dot_general(wc_ref[...], agg_ref[b], dn,
                         preferred_element_type=jnp.float32) + bc_ref[...]
    y_ref[b] = yb
    ym = jnp.where(valid, yb, 0.0)
    s = s + jnp.sum(ym, axis=1, keepdims=True)
    q = q + jnp.sum(ym * ym, axis=1, keepdims=True)
  ssum_ref[...] += s
  ssq_ref[...] += q


def _conv(pooled, agg, wp, wc, bp, bc):
  return pl.pallas_call(
      _conv_body,
      grid=(NQB,),
      in_specs=[
          pl.BlockSpec((B, QBLK, C), lambda i: (0, i, 0)),
          pl.BlockSpec((B, QBLK, C), lambda i: (0, i, 0)),
          pl.BlockSpec((C, C), lambda i: (0, 0)),
          pl.BlockSpec((C, C), lambda i: (0, 0)),
          pl.BlockSpec((C, 1), lambda i: (0, 0)),
          pl.BlockSpec((C, 1), lambda i: (0, 0)),
      ],
      out_specs=[
          pl.BlockSpec((B, C, QBLK), lambda i: (0, 0, i)),
          pl.BlockSpec((B, C, QBLK), lambda i: (0, 0, i)),
          pl.BlockSpec((C, 1), lambda i: (0, 0)),
          pl.BlockSpec((C, 1), lambda i: (0, 0)),
      ],
      out_shape=[
          jax.ShapeDtypeStruct((B, C, N_OUT_PAD), jnp.float32),
          jax.ShapeDtypeStruct((B, C, N_OUT_PAD), jnp.float32),
          jax.ShapeDtypeStruct((C, 1), jnp.float32),
          jax.ShapeDtypeStruct((C, 1), jnp.float32),
      ],
  )(pooled, agg, wp, wc, bp, bc)


def _bn_body(y_ref, ident_ref, ssum_ref, ssq_ref, g_ref, be_ref, out_ref):
  cnt = jnp.float32(B * N_OUT)
  mean = ssum_ref[...] / cnt
  var = ssq_ref[...] / cnt - mean * mean
  inv = 1.0 / jnp.sqrt(var + EPS_BN)
  for b in range(B):
    yb = y_ref[b]
    out_ref[b] = g_ref[...] * ((yb - mean) * inv) + be_ref[...] + ident_ref[b]


def _bn(y, ident, ssum, ssq, gamma, beta):
  return pl.pallas_call(
      _bn_body,
      grid=(NQB,),
      in_specs=[
          pl.BlockSpec((B, C, QBLK), lambda i: (0, 0, i)),
          pl.BlockSpec((B, C, QBLK), lambda i: (0, 0, i)),
          pl.BlockSpec((C, 1), lambda i: (0, 0)),
          pl.BlockSpec((C, 1), lambda i: (0, 0)),
          pl.BlockSpec((C, 1), lambda i: (0, 0)),
          pl.BlockSpec((C, 1), lambda i: (0, 0)),
      ],
      out_specs=[pl.BlockSpec((B, C, QBLK), lambda i: (0, 0, i))],
      out_shape=[jax.ShapeDtypeStruct((B, C, N_OUT_PAD), jnp.float32)],
  )(y, ident, ssum, ssq, gamma, beta)[0]


def kernel(x, in_vertices, out_vertices, W_pool, b_pool, W_conv, b_conv,
           gamma, beta):
  inv = jnp.pad(in_vertices, ((0, N_IN_PAD - N_IN), (0, 0)),
                constant_values=1e6)
  outv = jnp.pad(out_vertices, ((0, N_OUT_PAD - N_OUT), (0, 0)))
  qx = outv[:, 0:1]
  qy = outv[:, 1:2]
  qz = outv[:, 2:3]
  px = inv[:, 0].reshape(1, N_IN_PAD)
  py = inv[:, 1].reshape(1, N_IN_PAD)
  pz = inv[:, 2].reshape(1, N_IN_PAD)
  idx, w = _knn(qx, qy, qz, px, py, pz)

  xT = jnp.pad(jnp.transpose(x, (0, 2, 1)),
               ((0, 0), (0, N_IN_PAD - N_IN), (0, 0)))
  xT = xT.reshape(B * N_IN_PAD, C)
  pooled, agg = _sc_gather_pool(xT, idx, jnp.repeat(w, 16, axis=1))

  ident, y, ssum, ssq = _conv(pooled, agg, W_pool, W_conv,
                              b_pool.reshape(C, 1), b_conv.reshape(C, 1))
  out = _bn(y, ident, ssum, ssq, gamma.reshape(C, 1), beta.reshape(C, 1))
  return out[:, :, :N_OUT]


# speedup vs baseline: 2.4689x; 1.1436x over previous
import functools

import jax
import jax.numpy as jnp
from jax import lax
from jax.experimental import pallas as pl
from jax.experimental.pallas import tpu as pltpu
from jax.experimental.pallas import tpu_sc as plsc

B = 4
C = 128
N_IN = 50000
N_OUT = 12500
K = 16
SIGMA = 0.1
EPS_BN = 1e-5

N_IN_PAD = 50048
N_OUT_PAD = 12544
QBLK = 128
NQB = N_OUT_PAD // QBLK
ROWS_PER_W = N_OUT_PAD // 32
RBLK = 8
NRB = ROWS_PER_W // RBLK


GRP = 16
NGRP = N_IN_PAD // GRP
CAND = 32


def _knn1a_body(qx, qy, qz, px, py, pz, gi_ref):
  nch = 8
  ch = N_IN_PAD // nch
  qxv = qx[...]
  qyv = qy[...]
  qzv = qz[...]
  gms = []
  for c in range(nch):
    sl = pl.ds(c * ch, ch)
    d = ((qxv - px[:, sl]) ** 2 + (qyv - py[:, sl]) ** 2
         + (qzv - pz[:, sl]) ** 2)
    gms.append(jnp.min(d.reshape(QBLK, ch // GRP, GRP), axis=-1))
  gm = jnp.concatenate(gms, axis=1)
  lane = lax.broadcasted_iota(jnp.int32, (QBLK, NGRP), 1)
  inds = []
  for _ in range(K):
    a = jnp.argmin(gm, axis=1).astype(jnp.int32)
    inds.append(a[:, None])
    gm = jnp.where(lane == a[:, None], jnp.float32(jnp.inf), gm)
  gi_ref[...] = jnp.concatenate(inds, axis=1)


def _knn1a(qx, qy, qz, px, py, pz):
  return pl.pallas_call(
      _knn1a_body,
      grid=(NQB,),
      in_specs=[
          pl.BlockSpec((QBLK, 1), lambda i: (i, 0)),
          pl.BlockSpec((QBLK, 1), lambda i: (i, 0)),
          pl.BlockSpec((QBLK, 1), lambda i: (i, 0)),
          pl.BlockSpec((1, N_IN_PAD), lambda i: (0, 0)),
          pl.BlockSpec((1, N_IN_PAD), lambda i: (0, 0)),
          pl.BlockSpec((1, N_IN_PAD), lambda i: (0, 0)),
      ],
      out_specs=[
          pl.BlockSpec((QBLK, K), lambda i: (i, 0)),
      ],
      out_shape=[
          jax.ShapeDtypeStruct((N_OUT_PAD, K), jnp.int32),
      ],
  )(qx, qy, qz, px, py, pz)[0]


def _knn1b(gi, qexp, gall):
  mesh = plsc.VectorSubcoreMesh(core_axis_name="c", subcore_axis_name="s")

  @functools.partial(
      pl.kernel,
      mesh=mesh,
      out_type=[
          jax.ShapeDtypeStruct((N_OUT_PAD, K), jnp.int32),
          jax.ShapeDtypeStruct((N_OUT_PAD, K), jnp.float32),
      ],
      scratch_types=[
          pltpu.VMEM((ROWS_PER_W, K), jnp.int32),
          pltpu.VMEM((ROWS_PER_W, 48), jnp.float32),
          pltpu.VMEM((16, 128), jnp.float32),
          pltpu.VMEM((16,), jnp.int32),
          pltpu.VMEM((16,), jnp.float32),
          pltpu.SemaphoreType.DMA,
      ],
  )
  def kern(gi_hbm, qexp_hbm, gall_hbm,
           idx_hbm, w_hbm, gi_v, qv, gar,
           irow, wrow, sem):
    wid = lax.axis_index("s") * 2 + lax.axis_index("c")
    base = wid * ROWS_PER_W
    pltpu.sync_copy(gi_hbm.at[pl.ds(base, ROWS_PER_W)], gi_v)
    pltpu.sync_copy(qexp_hbm.at[pl.ds(base, ROWS_PER_W)], qv)
    lanes = lax.broadcasted_iota(jnp.int32, (16,), 0)

    def do_row(r, _):
      giv = gi_v[r]
      pltpu.async_copy(gall_hbm.at[giv], gar, sem).wait()
      qxv = qv[r, pl.ds(0, 16)]
      qyv = qv[r, pl.ds(16, 16)]
      qzv = qv[r, pl.ds(32, 16)]

      dvs, ivs = [], []
      for s in range(16):
        dx = qxv - gar[s, pl.ds(0, 16)]
        dy = qyv - gar[s, pl.ds(16, 16)]
        dz = qzv - gar[s, pl.ds(32, 16)]
        dvs.append(dx * dx + dy * dy + dz * dz)
        ivs.append(gar[s, pl.ds(48, 16)].astype(jnp.int32))

      big = jnp.float32(jnp.inf)
      bigi = jnp.int32(2 ** 30)
      Rk = jnp.zeros((16,), jnp.float32)
      Ri = jnp.zeros((16,), jnp.int32)
      for k in range(16):
        m = dvs[0]
        for s in range(1, 16):
          m = jnp.minimum(m, dvs[s])
        for sh in (1, 2, 4, 8):
          m = jnp.minimum(m, m.at[lanes ^ sh].get(mode="promise_in_bounds"))
        idc = jnp.full((16,), bigi, jnp.int32)
        for s in range(16):
          idc = jnp.minimum(idc, jnp.where(dvs[s] == m, ivs[s], bigi))
        for sh in (1, 2, 4, 8):
          idc = jnp.minimum(
              idc, idc.at[lanes ^ sh].get(mode="promise_in_bounds"))
        for s in range(16):
          hit = (dvs[s] == m) & (ivs[s] == idc)
          dvs[s] = jnp.where(hit, big, dvs[s])
        Rk = jnp.where(lanes == k, m, Rk)
        Ri = jnp.where(lanes == k, idc, Ri)

      wk = jnp.exp(-Rk / jnp.float32(2.0 * SIGMA * SIGMA))
      ws = wk
      for sh in (1, 2, 4, 8):
        ws = ws + ws.at[lanes ^ sh].get(mode="promise_in_bounds")
      wv = wk / (ws + 1e-12)
      irow[...] = Ri
      wrow[...] = wv
      pltpu.sync_copy(irow, idx_hbm.at[base + r])
      pltpu.sync_copy(wrow, w_hbm.at[base + r])
      return _

    lax.fori_loop(0, ROWS_PER_W, do_row, 0)

  return kern(gi, qexp, gall)


def _sc_gather_pool(xT, idx, wexp):
  mesh = plsc.VectorSubcoreMesh(core_axis_name="c", subcore_axis_name="s")

  @functools.partial(
      pl.kernel,
      mesh=mesh,
      out_type=[
          jax.ShapeDtypeStruct((B, N_OUT_PAD, C), jnp.float32),
          jax.ShapeDtypeStruct((B, N_OUT_PAD, C), jnp.float32),
      ],
      scratch_types=[
          pltpu.VMEM((ROWS_PER_W, K), jnp.int32),
          pltpu.VMEM((RBLK, K * 16), jnp.float32),
          pltpu.VMEM((B, K, C), jnp.float32),
          pltpu.VMEM((B, RBLK, C), jnp.float32),
          pltpu.VMEM((B, RBLK, C), jnp.float32),
          pltpu.SemaphoreType.DMA,
      ],
  )
  def kern(xT_hbm, idx_hbm, w_hbm, pooled_hbm, agg_hbm,
           idx_v, w_v, rows_v, poolblk, aggblk, sem):
    wid = lax.axis_index("s") * 2 + lax.axis_index("c")
    base = wid * ROWS_PER_W
    pltpu.sync_copy(idx_hbm.at[pl.ds(base, ROWS_PER_W)], idx_v)

    def do_block(blk, _):
      pltpu.sync_copy(w_hbm.at[pl.ds(base + blk * RBLK, RBLK)], w_v)

      def do_row(rr, _):
        r = blk * RBLK + rr
        iv = idx_v[r]
        cps = []
        for b in range(B):
          cps.append(pltpu.async_copy(
              xT_hbm.at[iv + b * N_IN_PAD], rows_v.at[b], sem))
        for cp in cps:
          cp.wait()
        wks = [w_v[rr, pl.ds(kk * 16, 16)] for kk in range(K)]
        for b in range(B):
          for g in range(C // 16):
            accp = rows_v[b, 0, pl.ds(g * 16, 16)]
            acca = wks[0] * accp
            for kk in range(1, K):
              row = rows_v[b, kk, pl.ds(g * 16, 16)]
              accp = accp + row
              acca = acca + wks[kk] * row
            poolblk[b, rr, pl.ds(g * 16, 16)] = accp * jnp.float32(1.0 / K)
            aggblk[b, rr, pl.ds(g * 16, 16)] = acca
        return _

      lax.fori_loop(0, RBLK, do_row, 0)
      for b in range(B):
        pltpu.sync_copy(poolblk.at[b],
                        pooled_hbm.at[b, pl.ds(base + blk * RBLK, RBLK)])
        pltpu.sync_copy(aggblk.at[b],
                        agg_hbm.at[b, pl.ds(base + blk * RBLK, RBLK)])
      return _

    lax.fori_loop(0, NRB, do_block, 0)

  return kern(xT, idx, wexp)


def _conv_body(pooled_ref, agg_ref, wp_ref, wc_ref, bp_ref, bc_ref,
               ident_ref, y_ref, ssum_ref, ssq_ref):
  i = pl.program_id(0)

  @pl.when(i == 0)
  def _():
    ssum_ref[...] = jnp.zeros((C, 1), jnp.float32)
    ssq_ref[...] = jnp.zeros((C, 1), jnp.float32)

  valid = (lax.broadcasted_iota(jnp.int32, (1, QBLK), 1) + i * QBLK) < N_OUT
  s = jnp.zeros((C, 1), jnp.float32)
  q = jnp.zeros((C, 1), jnp.float32)
  dn = (((1,), (1,)), ((), ()))
  for b in range(B):
    ib = lax.dot_general(wp_ref[...], pooled_ref[b], dn,
                         preferred_element_type=jnp.float32) + bp_ref[...]
    ident_ref[b] = ib
    yb = lax.dot_general(wc_ref[...], agg_ref[b], dn,
                         preferred_element_type=jnp.float32) + bc_ref[...]
    y_ref[b] = yb
    ym = jnp.where(valid, yb, 0.0)
    s = s + jnp.sum(ym, axis=1, keepdims=True)
    q = q + jnp.sum(ym * ym, axis=1, keepdims=True)
  ssum_ref[...] += s
  ssq_ref[...] += q


def _conv(pooled, agg, wp, wc, bp, bc):
  return pl.pallas_call(
      _conv_body,
      grid=(NQB,),
      in_specs=[
          pl.BlockSpec((B, QBLK, C), lambda i: (0, i, 0)),
          pl.BlockSpec((B, QBLK, C), lambda i: (0, i, 0)),
          pl.BlockSpec((C, C), lambda i: (0, 0)),
          pl.BlockSpec((C, C), lambda i: (0, 0)),
          pl.BlockSpec((C, 1), lambda i: (0, 0)),
          pl.BlockSpec((C, 1), lambda i: (0, 0)),
      ],
      out_specs=[
          pl.BlockSpec((B, C, QBLK), lambda i: (0, 0, i)),
          pl.BlockSpec((B, C, QBLK), lambda i: (0, 0, i)),
          pl.BlockSpec((C, 1), lambda i: (0, 0)),
          pl.BlockSpec((C, 1), lambda i: (0, 0)),
      ],
      out_shape=[
          jax.ShapeDtypeStruct((B, C, N_OUT_PAD), jnp.float32),
          jax.ShapeDtypeStruct((B, C, N_OUT_PAD), jnp.float32),
          jax.ShapeDtypeStruct((C, 1), jnp.float32),
          jax.ShapeDtypeStruct((C, 1), jnp.float32),
      ],
  )(pooled, agg, wp, wc, bp, bc)


def _bn_body(y_ref, ident_ref, ssum_ref, ssq_ref, g_ref, be_ref, out_ref):
  cnt = jnp.float32(B * N_OUT)
  mean = ssum_ref[...] / cnt
  var = ssq_ref[...] / cnt - mean * mean
  inv = 1.0 / jnp.sqrt(var + EPS_BN)
  for b in range(B):
    yb = y_ref[b]
    out_ref[b] = g_ref[...] * ((yb - mean) * inv) + be_ref[...] + ident_ref[b]


def _bn(y, ident, ssum, ssq, gamma, beta):
  return pl.pallas_call(
      _bn_body,
      grid=(NQB,),
      in_specs=[
          pl.BlockSpec((B, C, QBLK), lambda i: (0, 0, i)),
          pl.BlockSpec((B, C, QBLK), lambda i: (0, 0, i)),
          pl.BlockSpec((C, 1), lambda i: (0, 0)),
          pl.BlockSpec((C, 1), lambda i: (0, 0)),
          pl.BlockSpec((C, 1), lambda i: (0, 0)),
          pl.BlockSpec((C, 1), lambda i: (0, 0)),
      ],
      out_specs=[pl.BlockSpec((B, C, QBLK), lambda i: (0, 0, i))],
      out_shape=[jax.ShapeDtypeStruct((B, C, N_OUT_PAD), jnp.float32)],
  )(y, ident, ssum, ssq, gamma, beta)[0]


def kernel(x, in_vertices, out_vertices, W_pool, b_pool, W_conv, b_conv,
           gamma, beta):
  inv = jnp.pad(in_vertices, ((0, N_IN_PAD - N_IN), (0, 0)),
                constant_values=1e6)
  outv = jnp.pad(out_vertices, ((0, N_OUT_PAD - N_OUT), (0, 0)))
  qx = outv[:, 0:1]
  qy = outv[:, 1:2]
  qz = outv[:, 2:3]
  px = inv[:, 0].reshape(1, N_IN_PAD)
  py = inv[:, 1].reshape(1, N_IN_PAD)
  pz = inv[:, 2].reshape(1, N_IN_PAD)
  gi = _knn1a(qx, qy, qz, px, py, pz)
  qexp = jnp.concatenate(
      [jnp.repeat(qx, 16, axis=1), jnp.repeat(qy, 16, axis=1),
       jnp.repeat(qz, 16, axis=1)], axis=1)
  gall = jnp.concatenate(
      [inv[:, 0].reshape(NGRP, GRP), inv[:, 1].reshape(NGRP, GRP),
       inv[:, 2].reshape(NGRP, GRP),
       jnp.arange(N_IN_PAD, dtype=jnp.float32).reshape(NGRP, GRP),
       jnp.zeros((NGRP, 64), jnp.float32)], axis=1)
  idx, w = _knn1b(gi, qexp, gall)

  xT = jnp.pad(jnp.transpose(x, (0, 2, 1)),
               ((0, 0), (0, N_IN_PAD - N_IN), (0, 0)))
  xT = xT.reshape(B * N_IN_PAD, C)
  pooled, agg = _sc_gather_pool(xT, idx, jnp.repeat(w, 16, axis=1))

  ident, y, ssum, ssq = _conv(pooled, agg, W_pool, W_conv,
                              b_pool.reshape(C, 1), b_conv.reshape(C, 1))
  out = _bn(y, ident, ssum, ssq, gamma.reshape(C, 1), beta.reshape(C, 1))
  return out[:, :, :N_OUT]
